# Initial kernel scaffold; baseline (speedup 1.0000x reference)
import jax
import jax.numpy as jnp
from jax.experimental import pallas as pl


def kernel(h, edge_index):
    raise NotImplementedError("write your pallas kernel here")


# speedup vs baseline: 3.3574x; 3.3574x over previous
import functools

import jax
import jax.numpy as jnp
from jax import lax
from jax.experimental import pallas as pl
from jax.experimental.pallas import tpu as pltpu
from jax.experimental.pallas import tpu_sc as plsc

N_NODES = 10000
N_EDGES = 320000
D_FEAT = 128

NW = 32
PER_W = N_EDGES // NW
CHUNK = 80
NCHUNK = PER_W // CHUNK

LANES = 16
D_VECS = D_FEAT // LANES


def _sc_kernel_body(h_hbm, src_hbm, dst_hbm, out_hbm,
                    idx_s, idx_d, rows_s, rows_d, out_v, t16, sem):
    wid = lax.axis_index("s") * 2 + lax.axis_index("c")
    wbase = wid * PER_W
    lane_iota = lax.iota(jnp.int32, LANES)

    def chunk_body(c, carry):
        base = wbase + c * CHUNK
        pltpu.sync_copy(src_hbm.at[pl.ds(base, CHUNK)], idx_s)
        pltpu.sync_copy(dst_hbm.at[pl.ds(base, CHUNK)], idx_d)
        cp_s = pltpu.make_async_copy(h_hbm.at[idx_s], rows_s, sem)
        cp_d = pltpu.make_async_copy(h_hbm.at[idx_d], rows_d, sem)
        cp_s.start()
        cp_d.start()
        cp_s.wait()
        cp_d.wait()

        def group_body(g, carry2):
            gbase = g * LANES
            for j in range(LANES):
                e = gbase + j
                acc = rows_s[e, pl.ds(0, LANES)] * rows_d[e, pl.ds(0, LANES)]
                for k in range(1, D_VECS):
                    acc = acc + (rows_s[e, pl.ds(k * LANES, LANES)]
                                 * rows_d[e, pl.ds(k * LANES, LANES)])
                t16[pl.ds(j * LANES, LANES)] = acc
            row_base = lane_iota * LANES
            r = plsc.load_gather(t16, [row_base])
            for i in range(1, LANES):
                r = r + plsc.load_gather(t16, [row_base + i])
            out_v[pl.ds(gbase, LANES)] = r
            return carry2

        lax.fori_loop(0, CHUNK // LANES, group_body, 0)
        pltpu.sync_copy(out_v, out_hbm.at[pl.ds(base, CHUNK)])
        return carry

    lax.fori_loop(0, NCHUNK, chunk_body, 0)


@jax.jit
def _scores(h, src, dst):
    mesh = plsc.VectorSubcoreMesh(core_axis_name="c", subcore_axis_name="s")
    kfn = functools.partial(
        pl.kernel,
        mesh=mesh,
        compiler_params=pltpu.CompilerParams(needs_layout_passes=False),
        out_type=jax.ShapeDtypeStruct((N_EDGES,), jnp.float32),
        scratch_types=[
            pltpu.VMEM((CHUNK,), jnp.int32),
            pltpu.VMEM((CHUNK,), jnp.int32),
            pltpu.VMEM((CHUNK, D_FEAT), jnp.float32),
            pltpu.VMEM((CHUNK, D_FEAT), jnp.float32),
            pltpu.VMEM((CHUNK,), jnp.float32),
            pltpu.VMEM((LANES * LANES,), jnp.float32),
            pltpu.SemaphoreType.DMA,
        ],
    )(_sc_kernel_body)
    return kfn(h, src, dst)


def kernel(h, edge_index):
    src = edge_index[0].astype(jnp.int32)
    dst = edge_index[1].astype(jnp.int32)
    scores = _scores(h, src, dst)
    return scores.reshape(N_EDGES, 1)


# speedup vs baseline: 7.4897x; 2.2308x over previous
import functools

import jax
import jax.numpy as jnp
from jax import lax
from jax.experimental import pallas as pl
from jax.experimental.pallas import tpu as pltpu
from jax.experimental.pallas import tpu_sc as plsc

N_NODES = 10000
N_EDGES = 320000
D_FEAT = 128

NW = 32
PER_W = N_EDGES // NW
CHUNK = 80
NCHUNK = PER_W // CHUNK

LANES = 16
D_VECS = D_FEAT // LANES


def _sc_kernel_body(h_hbm, src_hbm, dst_hbm, out_hbm,
                    idx_s, idx_d, rows_s, rows_d, out_v, t16, sem0, sem1):
    wid = lax.axis_index("s") * 2 + lax.axis_index("c")
    wbase = wid * PER_W
    lane_iota = lax.iota(jnp.int32, LANES)

    pltpu.sync_copy(src_hbm.at[pl.ds(wbase, PER_W)], idx_s)
    pltpu.sync_copy(dst_hbm.at[pl.ds(wbase, PER_W)], idx_d)

    def gather_copies(c, p, sem):
        cp_s = pltpu.make_async_copy(
            h_hbm.at[idx_s.at[pl.ds(c * CHUNK, CHUNK)]], rows_s.at[p], sem)
        cp_d = pltpu.make_async_copy(
            h_hbm.at[idx_d.at[pl.ds(c * CHUNK, CHUNK)]], rows_d.at[p], sem)
        return cp_s, cp_d

    def start(c, p, sem):
        cp_s, cp_d = gather_copies(c, p, sem)
        cp_s.start()
        cp_d.start()

    def wait(c, p, sem):
        cp_s, cp_d = gather_copies(c, p, sem)
        cp_s.wait()
        cp_d.wait()

    start(0, 0, sem0)
    start(1, 1, sem1)

    def chunk_body(c, carry):
        p = lax.rem(c, 2)

        def do_chunk(p_const, sem_p):
            wait(c, p_const, sem_p)

            def group_body(g, carry2):
                gbase = g * LANES
                for j in range(LANES):
                    acc = (rows_s[p_const, gbase + j, pl.ds(0, LANES)]
                           * rows_d[p_const, gbase + j, pl.ds(0, LANES)])
                    for k in range(1, D_VECS):
                        acc = acc + (
                            rows_s[p_const, gbase + j, pl.ds(k * LANES, LANES)]
                            * rows_d[p_const, gbase + j, pl.ds(k * LANES, LANES)])
                    t16[pl.ds(j * LANES, LANES)] = acc
                row_base = lane_iota * LANES
                r = plsc.load_gather(t16, [row_base])
                for i in range(1, LANES):
                    r = r + plsc.load_gather(t16, [row_base + i])
                out_v[pl.ds(c * CHUNK + gbase, LANES)] = r
                return carry2

            lax.fori_loop(0, CHUNK // LANES, group_body, 0)

            @pl.when(c + 2 < NCHUNK)
            def _():
                start(c + 2, p_const, sem_p)

        @pl.when(p == 0)
        def _():
            do_chunk(0, sem0)

        @pl.when(p == 1)
        def _():
            do_chunk(1, sem1)

        return carry

    lax.fori_loop(0, NCHUNK, chunk_body, 0)
    pltpu.sync_copy(out_v, out_hbm.at[pl.ds(wbase, PER_W)])


@jax.jit
def _scores(h, src, dst):
    mesh = plsc.VectorSubcoreMesh(core_axis_name="c", subcore_axis_name="s")
    kfn = functools.partial(
        pl.kernel,
        mesh=mesh,
        compiler_params=pltpu.CompilerParams(needs_layout_passes=False),
        out_type=jax.ShapeDtypeStruct((N_EDGES,), jnp.float32),
        scratch_types=[
            pltpu.VMEM((PER_W,), jnp.int32),
            pltpu.VMEM((PER_W,), jnp.int32),
            pltpu.VMEM((2, CHUNK, D_FEAT), jnp.float32),
            pltpu.VMEM((2, CHUNK, D_FEAT), jnp.float32),
            pltpu.VMEM((PER_W,), jnp.float32),
            pltpu.VMEM((LANES * LANES,), jnp.float32),
            pltpu.SemaphoreType.DMA,
            pltpu.SemaphoreType.DMA,
        ],
    )(_sc_kernel_body)
    return kfn(h, src, dst)


def kernel(h, edge_index):
    src = edge_index[0].astype(jnp.int32)
    dst = edge_index[1].astype(jnp.int32)
    scores = _scores(h, src, dst)
    return scores.reshape(N_EDGES, 1)
